# baseline (speedup 1.0000x reference)
import math

import jax
import jax.numpy as jnp
from jax.experimental import pallas as pl


def _proj_body(m0_ref, m1_ref, wq_ref, wk_ref, qt_ref, kt_ref):
    dn = (((1,), (0,)), ((), ()))
    m0 = m0_ref[...].astype(jnp.float32)
    m1 = m1_ref[...].astype(jnp.float32)
    q0 = jax.lax.dot_general(wq_ref[...], m0, dn,
                             preferred_element_type=jnp.float32)
    q1 = jax.lax.dot_general(wq_ref[...], m1, dn,
                             preferred_element_type=jnp.float32)
    k0 = jax.lax.dot_general(wk_ref[...], m0, dn,
                             preferred_element_type=jnp.float32)
    k1 = jax.lax.dot_general(wk_ref[...], m1, dn,
                             preferred_element_type=jnp.float32)
    qt_ref[...] = jnp.concatenate([q0, q1], axis=0)
    kt_ref[...] = jnp.concatenate([k0, k1], axis=0)


def _attn_body(qt_ref, kt_ref, out_ref):
    s = jax.lax.dot_general(qt_ref[...], kt_ref[...], (((0,), (0,)), ((), ())),
                            preferred_element_type=jnp.float32)
    e = jnp.exp(s * (1.0 / math.sqrt(32.0)))
    out_ref[...] = e * (1.0 / jnp.sum(e, axis=-1, keepdims=True))


def kernel(x, W_Q, W_K):
    B, F, N, T = x.shape
    D = T * F
    P = W_Q.shape[0]
    C = B * P
    m = jnp.transpose(x, (0, 3, 1, 2)).reshape(B, D, N).astype(jnp.bfloat16)
    m0, m1 = m[0], m[1]

    R1 = 1024
    qt, kt = pl.pallas_call(
        _proj_body,
        grid=(N // R1,),
        in_specs=[
            pl.BlockSpec((D, R1), lambda i: (0, i)),
            pl.BlockSpec((D, R1), lambda i: (0, i)),
            pl.BlockSpec((P, D), lambda i: (0, 0)),
            pl.BlockSpec((P, D), lambda i: (0, 0)),
        ],
        out_specs=[
            pl.BlockSpec((C, R1), lambda i: (0, i)),
            pl.BlockSpec((C, R1), lambda i: (0, i)),
        ],
        out_shape=[
            jax.ShapeDtypeStruct((C, N), jnp.float32),
            jax.ShapeDtypeStruct((C, N), jnp.float32),
        ],
    )(m0, m1, W_Q, W_K)

    R2 = 512
    out = pl.pallas_call(
        _attn_body,
        grid=(N // R2,),
        in_specs=[
            pl.BlockSpec((C, R2), lambda i: (0, i)),
            pl.BlockSpec((C, N), lambda i: (0, 0)),
        ],
        out_specs=pl.BlockSpec((R2, N), lambda i: (i, 0)),
        out_shape=jax.ShapeDtypeStruct((N, N), jnp.float32),
    )(qt, kt)
    return out


# speedup vs baseline: 1.7051x; 1.7051x over previous
import math

import jax
import jax.numpy as jnp
from jax.experimental import pallas as pl
from jax.experimental.pallas import tpu as pltpu


def _fused_body(m_ref, wq_ref, wk_ref, out_ref, qt_s, kt_s,
                *, G1, R1, R2):
    i = pl.program_id(0)

    @pl.when(i < G1)
    def _proj_phase():
        dn = (((1,), (0,)), ((), ()))
        q0 = jax.lax.dot_general(wq_ref[...], m_ref[0], dn,
                                 preferred_element_type=jnp.float32)
        q1 = jax.lax.dot_general(wq_ref[...], m_ref[1], dn,
                                 preferred_element_type=jnp.float32)
        k0 = jax.lax.dot_general(wk_ref[...], m_ref[0], dn,
                                 preferred_element_type=jnp.float32)
        k1 = jax.lax.dot_general(wk_ref[...], m_ref[1], dn,
                                 preferred_element_type=jnp.float32)
        col = i * R1
        qt_s[:, pl.ds(col, R1)] = jnp.concatenate([q0, q1], axis=0)
        kt_s[:, pl.ds(col, R1)] = jnp.concatenate([k0, k1], axis=0)

    @pl.when(i >= G1)
    def _attn_phase():
        j = i - G1
        qt_blk = qt_s[:, pl.ds(j * R2, R2)]
        s = jax.lax.dot_general(qt_blk, kt_s[...], (((0,), (0,)), ((), ())),
                                preferred_element_type=jnp.float32)
        e = jnp.exp(s * (1.0 / math.sqrt(32.0)))
        out_ref[...] = e * (1.0 / jnp.sum(e, axis=-1, keepdims=True))


def kernel(x, W_Q, W_K):
    B, F, N, T = x.shape
    D = T * F
    P = W_Q.shape[0]
    C = B * P
    m = jnp.transpose(x, (0, 3, 1, 2)).reshape(B, D, N)

    R1 = 1024
    R2 = 512
    G1 = N // R1
    G2 = N // R2
    import functools
    body = functools.partial(_fused_body, G1=G1, R1=R1, R2=R2)
    out = pl.pallas_call(
        body,
        grid=(G1 + G2,),
        in_specs=[
            pl.BlockSpec((B, D, R1), lambda i: (0, 0, jnp.minimum(i, G1 - 1))),
            pl.BlockSpec((P, D), lambda i: (0, 0)),
            pl.BlockSpec((P, D), lambda i: (0, 0)),
        ],
        out_specs=pl.BlockSpec((R2, N), lambda i: (jnp.maximum(i - G1, 0), 0)),
        out_shape=jax.ShapeDtypeStruct((N, N), jnp.float32),
        scratch_shapes=[
            pltpu.VMEM((C, N), jnp.float32),
            pltpu.VMEM((C, N), jnp.float32),
        ],
    )(m, W_Q, W_K)
    return out
